# baseline (speedup 1.0000x reference)
import jax
import jax.numpy as jnp
import numpy as np
from jax.experimental import pallas as pl

_BN_SCALE = np.float32(1.0 / np.sqrt(1.0 + 1e-5))

_BM = 200
_BM_TAIL = 2000


def _proj_body(x_ref, xa_ref, ws_ref, wf_ref, wa_ref, ps_ref, pf_ref, q_ref):
    x = x_ref[...]
    ps_ref[...] = jnp.dot(x, ws_ref[...], preferred_element_type=jnp.float32)
    pf_ref[...] = jnp.dot(x, wf_ref[...], preferred_element_type=jnp.float32)
    q_ref[...] = jnp.dot(xa_ref[...], wa_ref[...], preferred_element_type=jnp.float32)


def _layer1_body(adj_ref, p_ref, b1_ref, w2_ref, u_ref):
    h = jnp.dot(adj_ref[...], p_ref[...], preferred_element_type=jnp.float32)
    h = jnp.maximum(h + b1_ref[...], 0.0)
    u_ref[...] = jnp.dot(h, w2_ref[...], preferred_element_type=jnp.float32)


def _layer2_body(adj_ref, u_ref, b2_ref, emb_ref, com_ref):
    o = jnp.dot(adj_ref[...], u_ref[...], preferred_element_type=jnp.float32)
    o = o + b2_ref[...]
    emb_ref[...] = o[:, :32]
    com_ref[...] = o[:, 32:]


def _gcn_pair(adj, p, b1cat, w2bd, b2cat):
    n = adj.shape[0]
    hcat = p.shape[1]
    u = pl.pallas_call(
        _layer1_body,
        grid=(n // _BM,),
        in_specs=[
            pl.BlockSpec((_BM, n), lambda i: (i, 0)),
            pl.BlockSpec((n, hcat), lambda i: (0, 0)),
            pl.BlockSpec((1, hcat), lambda i: (0, 0)),
            pl.BlockSpec((hcat, 64), lambda i: (0, 0)),
        ],
        out_specs=pl.BlockSpec((_BM, 64), lambda i: (i, 0)),
        out_shape=jax.ShapeDtypeStruct((n, 64), jnp.float32),
    )(adj, p, b1cat, w2bd)
    emb, com = pl.pallas_call(
        _layer2_body,
        grid=(n // _BM,),
        in_specs=[
            pl.BlockSpec((_BM, n), lambda i: (i, 0)),
            pl.BlockSpec((n, 64), lambda i: (0, 0)),
            pl.BlockSpec((1, 64), lambda i: (0, 0)),
        ],
        out_specs=[
            pl.BlockSpec((_BM, 32), lambda i: (i, 0)),
            pl.BlockSpec((_BM, 32), lambda i: (i, 0)),
        ],
        out_shape=[
            jax.ShapeDtypeStruct((n, 32), jnp.float32),
            jax.ShapeDtypeStruct((n, 32), jnp.float32),
        ],
    )(adj, u, b2cat)
    return emb, com


def _tail_body(e1_ref, e2_ref, e3_ref, c1_ref, c2_ref, c3_ref,
               attw1_ref, attb1_ref, attw2_ref,
               mlpw_ref, mlpb_ref,
               decw_ref, decb_ref, decg_ref, decbeta_ref,
               piw_ref, pib_ref, dispw_ref, dispb_ref, meanw_ref, meanb_ref,
               cpw_ref, cpb_ref, cpg_ref, cpbeta_ref, ccw_ref, ccb_ref,
               emb_ref, pi_ref, disp_ref, mean_ref,
               y1_ref, p1_ref, y2_ref, p2_ref, y3_ref, p3_ref,
               y4_ref, p4_ref, y_ref, pz_ref):
    e1 = e1_ref[...]
    e2 = e2_ref[...]
    e3 = e3_ref[...]
    com = e1 + e2 + e3

    aw1 = attw1_ref[...]
    ab1 = attb1_ref[...]
    aw2 = attw2_ref[...]

    def att_score(zz):
        t = jnp.tanh(jnp.dot(zz, aw1, preferred_element_type=jnp.float32) + ab1)
        return jnp.sum(t * aw2, axis=1, keepdims=True)

    z1 = e1
    z2 = com * np.float32(1.0 / 3.0)
    z3 = e2
    z4 = e3
    w1 = att_score(z1)
    w2 = att_score(z2)
    w3 = att_score(z3)
    w4 = att_score(z4)
    m = jnp.maximum(jnp.maximum(w1, w2), jnp.maximum(w3, w4))
    x1 = jnp.exp(w1 - m)
    x2 = jnp.exp(w2 - m)
    x3 = jnp.exp(w3 - m)
    x4 = jnp.exp(w4 - m)
    s = x1 + x2 + x3 + x4
    emb = (x1 * z1 + x2 * z2 + x3 * z3 + x4 * z4) / s
    emb = jnp.dot(emb, mlpw_ref[...], preferred_element_type=jnp.float32) + mlpb_ref[...]
    emb_ref[...] = emb

    h = jnp.dot(emb, decw_ref[...], preferred_element_type=jnp.float32) + decb_ref[...]
    h = jnp.maximum(h * _BN_SCALE * decg_ref[...] + decbeta_ref[...], 0.0)
    pi_ref[...] = jax.nn.sigmoid(
        jnp.dot(h, piw_ref[...], preferred_element_type=jnp.float32) + pib_ref[...])
    dv = jnp.dot(h, dispw_ref[...], preferred_element_type=jnp.float32) + dispb_ref[...]
    disp_ref[...] = jnp.clip(jax.nn.softplus(dv), 0.0001, 10000.0)
    mv = jnp.dot(h, meanw_ref[...], preferred_element_type=jnp.float32) + meanb_ref[...]
    mean_ref[...] = jnp.clip(jnp.exp(mv), 1e-05, 1000000.0)

    cpw = cpw_ref[...]
    cpb = cpb_ref[...]
    cpg = cpg_ref[...]
    cpbeta = cpbeta_ref[...]
    ccw = ccw_ref[...]
    ccb = ccb_ref[...]

    def cluster(v, y_out, p_out):
        zz = jnp.dot(v, cpw, preferred_element_type=jnp.float32) + cpb
        zz = jnp.maximum(zz * _BN_SCALE * cpg + cpbeta, 0.0)
        logits = jnp.dot(zz, ccw, preferred_element_type=jnp.float32) + ccb
        logits = logits - jnp.max(logits, axis=1, keepdims=True)
        ex = jnp.exp(logits)
        y_out[...] = ex / jnp.sum(ex, axis=1, keepdims=True)
        p_out[...] = zz

    cluster(c1_ref[...], y1_ref, p1_ref)
    cluster(c2_ref[...], y2_ref, p2_ref)
    cluster(c3_ref[...], y3_ref, p3_ref)
    cluster(com, y4_ref, p4_ref)
    cluster(emb, y_ref, pz_ref)


def kernel(x, x_aug, sadj, fadj, stg, params):
    p = params
    n, nfeat = x.shape

    def r2(v):
        return v.reshape(1, -1)

    ws = jnp.concatenate([p['sgcn_W1'], p['cgcn_W1']], axis=1)
    wf = jnp.concatenate([p['fgcn_W1'], p['cgcn_W1']], axis=1)
    wa = jnp.concatenate([p['agcn_W1'], p['cgcn_W1']], axis=1)
    h1 = p['sgcn_W1'].shape[1]

    def bd(wb, wc):
        z = jnp.zeros((2 * h1, 64), jnp.float32)
        z = z.at[:h1, :32].set(wb)
        z = z.at[h1:, 32:].set(wc)
        return z

    w2_s = bd(p['sgcn_W2'], p['cgcn_W2'])
    w2_f = bd(p['fgcn_W2'], p['cgcn_W2'])
    w2_a = bd(p['agcn_W2'], p['cgcn_W2'])
    b1_s = r2(jnp.concatenate([p['sgcn_b1'], p['cgcn_b1']]))
    b1_f = r2(jnp.concatenate([p['fgcn_b1'], p['cgcn_b1']]))
    b1_a = r2(jnp.concatenate([p['agcn_b1'], p['cgcn_b1']]))
    b2_s = r2(jnp.concatenate([p['sgcn_b2'], p['cgcn_b2']]))
    b2_f = r2(jnp.concatenate([p['fgcn_b2'], p['cgcn_b2']]))
    b2_a = r2(jnp.concatenate([p['agcn_b2'], p['cgcn_b2']]))

    hcat = 2 * h1
    ps, pf, q = pl.pallas_call(
        _proj_body,
        grid=(n // _BM_TAIL,),
        in_specs=[
            pl.BlockSpec((_BM_TAIL, nfeat), lambda i: (i, 0)),
            pl.BlockSpec((_BM_TAIL, nfeat), lambda i: (i, 0)),
            pl.BlockSpec((nfeat, hcat), lambda i: (0, 0)),
            pl.BlockSpec((nfeat, hcat), lambda i: (0, 0)),
            pl.BlockSpec((nfeat, hcat), lambda i: (0, 0)),
        ],
        out_specs=[pl.BlockSpec((_BM_TAIL, hcat), lambda i: (i, 0))] * 3,
        out_shape=[jax.ShapeDtypeStruct((n, hcat), jnp.float32)] * 3,
    )(x, x_aug, ws, wf, wa)

    emb1, com1 = _gcn_pair(sadj, ps, b1_s, w2_s, b2_s)
    emb2, com2 = _gcn_pair(fadj, pf, b1_f, w2_f, b2_f)
    emb3, com3 = _gcn_pair(stg, q, b1_a, w2_a, b2_a)

    nclass = p['cc_W'].shape[1]
    row_spec = lambda w: pl.BlockSpec((_BM_TAIL, w), lambda i: (i, 0))
    const_spec = lambda a: pl.BlockSpec(a.shape, lambda i: (0,) * a.ndim)

    weights = [
        p['att_W1'], r2(p['att_b1']), r2(p['att_W2'][:, 0]),
        p['mlp_W'], r2(p['mlp_b']),
        p['dec_W'], r2(p['dec_b']), r2(p['dec_g']), r2(p['dec_beta']),
        p['pi_W'], r2(p['pi_b']), p['disp_W'], r2(p['disp_b']),
        p['mean_W'], r2(p['mean_b']),
        p['cp_W'], r2(p['cp_b']), r2(p['cp_g']), r2(p['cp_beta']),
        p['cc_W'], r2(p['cc_b']),
    ]
    outs = pl.pallas_call(
        _tail_body,
        grid=(n // _BM_TAIL,),
        in_specs=[row_spec(32)] * 6 + [const_spec(w) for w in weights],
        out_specs=(
            [row_spec(32)] + [row_spec(nfeat)] * 3
            + [row_spec(nclass), row_spec(32)] * 5
        ),
        out_shape=(
            [jax.ShapeDtypeStruct((n, 32), jnp.float32)]
            + [jax.ShapeDtypeStruct((n, nfeat), jnp.float32)] * 3
            + [jax.ShapeDtypeStruct((n, nclass), jnp.float32),
               jax.ShapeDtypeStruct((n, 32), jnp.float32)] * 5
        ),
    )(emb1, emb2, emb3, com1, com2, com3, *weights)
    (emb, pi, disp, mean, y1, p1, y2, p2, y3, p3, y4, p4, y, pz) = outs
    return (com1, com2, com3, emb, pi, disp, mean,
            y1, p1, y2, p2, y3, p3, y4, p4, y, pz)


# speedup vs baseline: 1.0315x; 1.0315x over previous
import jax
import jax.numpy as jnp
import numpy as np
from jax.experimental import pallas as pl
from jax.experimental.pallas import tpu as pltpu

_BN_SCALE = np.float32(1.0 / np.sqrt(1.0 + 1e-5))

_BM = 400
_BM_TAIL = 2000


def _proj_body(x_ref, xa_ref, ws_ref, wf_ref, wa_ref, ps_ref, pf_ref, q_ref):
    x = x_ref[...]
    ps_ref[...] = jnp.dot(x, ws_ref[...], preferred_element_type=jnp.float32)
    pf_ref[...] = jnp.dot(x, wf_ref[...], preferred_element_type=jnp.float32)
    q_ref[...] = jnp.dot(xa_ref[...], wa_ref[...], preferred_element_type=jnp.float32)


def _gcn_body(adj_ref, p_ref, b1_ref, w2_ref, b2_ref, emb_ref, com_ref, u_scr):
    ph = pl.program_id(0)
    i = pl.program_id(1)

    @pl.when(ph == 0)
    def _layer1():
        h = jnp.dot(adj_ref[...], p_ref[...], preferred_element_type=jnp.float32)
        h = jnp.maximum(h + b1_ref[...], 0.0)
        u_scr[pl.ds(i * _BM, _BM), :] = jnp.dot(
            h, w2_ref[...], preferred_element_type=jnp.float32)

    @pl.when(ph == 1)
    def _layer2():
        o = jnp.dot(adj_ref[...], u_scr[...], preferred_element_type=jnp.float32)
        o = o + b2_ref[...]
        emb_ref[...] = o[:, :32]
        com_ref[...] = o[:, 32:]


def _gcn_pair(adj, p, b1cat, w2bd, b2cat):
    n = adj.shape[0]
    hcat = p.shape[1]
    emb, com = pl.pallas_call(
        _gcn_body,
        grid=(2, n // _BM),
        in_specs=[
            pl.BlockSpec((_BM, n), lambda ph, i: (i, 0)),
            pl.BlockSpec((n, hcat), lambda ph, i: (0, 0)),
            pl.BlockSpec((1, hcat), lambda ph, i: (0, 0)),
            pl.BlockSpec((hcat, 64), lambda ph, i: (0, 0)),
            pl.BlockSpec((1, 64), lambda ph, i: (0, 0)),
        ],
        out_specs=[
            pl.BlockSpec((_BM, 32), lambda ph, i: (i, 0)),
            pl.BlockSpec((_BM, 32), lambda ph, i: (i, 0)),
        ],
        out_shape=[
            jax.ShapeDtypeStruct((n, 32), jnp.float32),
            jax.ShapeDtypeStruct((n, 32), jnp.float32),
        ],
        scratch_shapes=[pltpu.VMEM((n, 64), jnp.float32)],
    )(adj, p, b1cat, w2bd, b2cat)
    return emb, com


def _tail_body(e1_ref, e2_ref, e3_ref, c1_ref, c2_ref, c3_ref,
               attw1_ref, attb1_ref, attw2_ref,
               mlpw_ref, mlpb_ref,
               decw_ref, decb_ref, decg_ref, decbeta_ref,
               piw_ref, pib_ref, dispw_ref, dispb_ref, meanw_ref, meanb_ref,
               cpw_ref, cpb_ref, cpg_ref, cpbeta_ref, ccw_ref, ccb_ref,
               emb_ref, pi_ref, disp_ref, mean_ref,
               y1_ref, p1_ref, y2_ref, p2_ref, y3_ref, p3_ref,
               y4_ref, p4_ref, y_ref, pz_ref):
    e1 = e1_ref[...]
    e2 = e2_ref[...]
    e3 = e3_ref[...]
    com = e1 + e2 + e3

    aw1 = attw1_ref[...]
    ab1 = attb1_ref[...]
    aw2 = attw2_ref[...]

    def att_score(zz):
        t = jnp.tanh(jnp.dot(zz, aw1, preferred_element_type=jnp.float32) + ab1)
        return jnp.sum(t * aw2, axis=1, keepdims=True)

    z1 = e1
    z2 = com * np.float32(1.0 / 3.0)
    z3 = e2
    z4 = e3
    w1 = att_score(z1)
    w2 = att_score(z2)
    w3 = att_score(z3)
    w4 = att_score(z4)
    m = jnp.maximum(jnp.maximum(w1, w2), jnp.maximum(w3, w4))
    x1 = jnp.exp(w1 - m)
    x2 = jnp.exp(w2 - m)
    x3 = jnp.exp(w3 - m)
    x4 = jnp.exp(w4 - m)
    s = x1 + x2 + x3 + x4
    emb = (x1 * z1 + x2 * z2 + x3 * z3 + x4 * z4) / s
    emb = jnp.dot(emb, mlpw_ref[...], preferred_element_type=jnp.float32) + mlpb_ref[...]
    emb_ref[...] = emb

    h = jnp.dot(emb, decw_ref[...], preferred_element_type=jnp.float32) + decb_ref[...]
    h = jnp.maximum(h * _BN_SCALE * decg_ref[...] + decbeta_ref[...], 0.0)
    pi_ref[...] = jax.nn.sigmoid(
        jnp.dot(h, piw_ref[...], preferred_element_type=jnp.float32) + pib_ref[...])
    dv = jnp.dot(h, dispw_ref[...], preferred_element_type=jnp.float32) + dispb_ref[...]
    disp_ref[...] = jnp.clip(jax.nn.softplus(dv), 0.0001, 10000.0)
    mv = jnp.dot(h, meanw_ref[...], preferred_element_type=jnp.float32) + meanb_ref[...]
    mean_ref[...] = jnp.clip(jnp.exp(mv), 1e-05, 1000000.0)

    cpw = cpw_ref[...]
    cpb = cpb_ref[...]
    cpg = cpg_ref[...]
    cpbeta = cpbeta_ref[...]
    ccw = ccw_ref[...]
    ccb = ccb_ref[...]

    def cluster(v, y_out, p_out):
        zz = jnp.dot(v, cpw, preferred_element_type=jnp.float32) + cpb
        zz = jnp.maximum(zz * _BN_SCALE * cpg + cpbeta, 0.0)
        logits = jnp.dot(zz, ccw, preferred_element_type=jnp.float32) + ccb
        logits = logits - jnp.max(logits, axis=1, keepdims=True)
        ex = jnp.exp(logits)
        y_out[...] = ex / jnp.sum(ex, axis=1, keepdims=True)
        p_out[...] = zz

    cluster(c1_ref[...], y1_ref, p1_ref)
    cluster(c2_ref[...], y2_ref, p2_ref)
    cluster(c3_ref[...], y3_ref, p3_ref)
    cluster(com, y4_ref, p4_ref)
    cluster(emb, y_ref, pz_ref)


def kernel(x, x_aug, sadj, fadj, stg, params):
    p = params
    n, nfeat = x.shape

    def r2(v):
        return v.reshape(1, -1)

    ws = jnp.concatenate([p['sgcn_W1'], p['cgcn_W1']], axis=1)
    wf = jnp.concatenate([p['fgcn_W1'], p['cgcn_W1']], axis=1)
    wa = jnp.concatenate([p['agcn_W1'], p['cgcn_W1']], axis=1)
    h1 = p['sgcn_W1'].shape[1]

    def bd(wb, wc):
        z = jnp.zeros((2 * h1, 64), jnp.float32)
        z = z.at[:h1, :32].set(wb)
        z = z.at[h1:, 32:].set(wc)
        return z

    w2_s = bd(p['sgcn_W2'], p['cgcn_W2'])
    w2_f = bd(p['fgcn_W2'], p['cgcn_W2'])
    w2_a = bd(p['agcn_W2'], p['cgcn_W2'])
    b1_s = r2(jnp.concatenate([p['sgcn_b1'], p['cgcn_b1']]))
    b1_f = r2(jnp.concatenate([p['fgcn_b1'], p['cgcn_b1']]))
    b1_a = r2(jnp.concatenate([p['agcn_b1'], p['cgcn_b1']]))
    b2_s = r2(jnp.concatenate([p['sgcn_b2'], p['cgcn_b2']]))
    b2_f = r2(jnp.concatenate([p['fgcn_b2'], p['cgcn_b2']]))
    b2_a = r2(jnp.concatenate([p['agcn_b2'], p['cgcn_b2']]))

    hcat = 2 * h1
    ps, pf, q = pl.pallas_call(
        _proj_body,
        grid=(n // _BM_TAIL,),
        in_specs=[
            pl.BlockSpec((_BM_TAIL, nfeat), lambda i: (i, 0)),
            pl.BlockSpec((_BM_TAIL, nfeat), lambda i: (i, 0)),
            pl.BlockSpec((nfeat, hcat), lambda i: (0, 0)),
            pl.BlockSpec((nfeat, hcat), lambda i: (0, 0)),
            pl.BlockSpec((nfeat, hcat), lambda i: (0, 0)),
        ],
        out_specs=[pl.BlockSpec((_BM_TAIL, hcat), lambda i: (i, 0))] * 3,
        out_shape=[jax.ShapeDtypeStruct((n, hcat), jnp.float32)] * 3,
    )(x, x_aug, ws, wf, wa)

    emb1, com1 = _gcn_pair(sadj, ps, b1_s, w2_s, b2_s)
    emb2, com2 = _gcn_pair(fadj, pf, b1_f, w2_f, b2_f)
    emb3, com3 = _gcn_pair(stg, q, b1_a, w2_a, b2_a)

    nclass = p['cc_W'].shape[1]
    row_spec = lambda w: pl.BlockSpec((_BM_TAIL, w), lambda i: (i, 0))
    const_spec = lambda a: pl.BlockSpec(a.shape, lambda i: (0,) * a.ndim)

    weights = [
        p['att_W1'], r2(p['att_b1']), r2(p['att_W2'][:, 0]),
        p['mlp_W'], r2(p['mlp_b']),
        p['dec_W'], r2(p['dec_b']), r2(p['dec_g']), r2(p['dec_beta']),
        p['pi_W'], r2(p['pi_b']), p['disp_W'], r2(p['disp_b']),
        p['mean_W'], r2(p['mean_b']),
        p['cp_W'], r2(p['cp_b']), r2(p['cp_g']), r2(p['cp_beta']),
        p['cc_W'], r2(p['cc_b']),
    ]
    outs = pl.pallas_call(
        _tail_body,
        grid=(n // _BM_TAIL,),
        in_specs=[row_spec(32)] * 6 + [const_spec(w) for w in weights],
        out_specs=(
            [row_spec(32)] + [row_spec(nfeat)] * 3
            + [row_spec(nclass), row_spec(32)] * 5
        ),
        out_shape=(
            [jax.ShapeDtypeStruct((n, 32), jnp.float32)]
            + [jax.ShapeDtypeStruct((n, nfeat), jnp.float32)] * 3
            + [jax.ShapeDtypeStruct((n, nclass), jnp.float32),
               jax.ShapeDtypeStruct((n, 32), jnp.float32)] * 5
        ),
    )(emb1, emb2, emb3, com1, com2, com3, *weights)
    (emb, pi, disp, mean, y1, p1, y2, p2, y3, p3, y4, p4, y, pz) = outs
    return (com1, com2, com3, emb, pi, disp, mean,
            y1, p1, y2, p2, y3, p3, y4, p4, y, pz)


# speedup vs baseline: 1.0338x; 1.0022x over previous
import jax
import jax.numpy as jnp
import numpy as np
from jax.experimental import pallas as pl
from jax.experimental.pallas import tpu as pltpu

_BN_SCALE = np.float32(1.0 / np.sqrt(1.0 + 1e-5))

_BM = 400
_BM_TAIL = 2000


def _proj_body(x_ref, xa_ref, ws_ref, wf_ref, wa_ref, ps_ref, pf_ref, q_ref):
    x = x_ref[...]
    ps_ref[...] = jnp.dot(x, ws_ref[...], preferred_element_type=jnp.float32)
    pf_ref[...] = jnp.dot(x, wf_ref[...], preferred_element_type=jnp.float32)
    q_ref[...] = jnp.dot(xa_ref[...], wa_ref[...], preferred_element_type=jnp.float32)


def _gcn_body(adj_ref, p_ref, b1_ref, w2_ref, b2_ref, emb_ref, com_ref, u_scr):
    ph = pl.program_id(0)
    i = pl.program_id(1)

    @pl.when(ph == 0)
    def _layer1():
        h = jnp.dot(adj_ref[...], p_ref[...], preferred_element_type=jnp.float32)
        h = jnp.maximum(h + b1_ref[...], 0.0)
        u_scr[pl.ds(i * _BM, _BM), :] = jnp.dot(
            h, w2_ref[...], preferred_element_type=jnp.float32)

    @pl.when(ph == 1)
    def _layer2():
        o = jnp.dot(adj_ref[...], u_scr[...], preferred_element_type=jnp.float32)
        o = o + b2_ref[...]
        emb_ref[...] = o[:, :32]
        com_ref[...] = o[:, 32:]


def _gcn_pair(adj, p, b1cat, w2bd, b2cat):
    n = adj.shape[0]
    hcat = p.shape[1]
    nsteps = n // _BM

    def rowmap(ph, i):
        return (jnp.where(ph == 0, i, nsteps - 1 - i), 0)

    emb, com = pl.pallas_call(
        _gcn_body,
        grid=(2, nsteps),
        in_specs=[
            pl.BlockSpec((_BM, n), rowmap),
            pl.BlockSpec((n, hcat), lambda ph, i: (0, 0)),
            pl.BlockSpec((1, hcat), lambda ph, i: (0, 0)),
            pl.BlockSpec((hcat, 64), lambda ph, i: (0, 0)),
            pl.BlockSpec((1, 64), lambda ph, i: (0, 0)),
        ],
        out_specs=[
            pl.BlockSpec((_BM, 32), rowmap),
            pl.BlockSpec((_BM, 32), rowmap),
        ],
        out_shape=[
            jax.ShapeDtypeStruct((n, 32), jnp.float32),
            jax.ShapeDtypeStruct((n, 32), jnp.float32),
        ],
        scratch_shapes=[pltpu.VMEM((n, 64), jnp.float32)],
    )(adj, p, b1cat, w2bd, b2cat)
    return emb, com


def _tail_body(e1_ref, e2_ref, e3_ref, c1_ref, c2_ref, c3_ref,
               attw1_ref, attb1_ref, attw2_ref,
               mlpw_ref, mlpb_ref,
               decw_ref, decb_ref, decg_ref, decbeta_ref,
               piw_ref, pib_ref, dispw_ref, dispb_ref, meanw_ref, meanb_ref,
               cpw_ref, cpb_ref, cpg_ref, cpbeta_ref, ccw_ref, ccb_ref,
               emb_ref, pi_ref, disp_ref, mean_ref,
               y1_ref, p1_ref, y2_ref, p2_ref, y3_ref, p3_ref,
               y4_ref, p4_ref, y_ref, pz_ref):
    e1 = e1_ref[...]
    e2 = e2_ref[...]
    e3 = e3_ref[...]
    com = e1 + e2 + e3

    aw1 = attw1_ref[...]
    ab1 = attb1_ref[...]
    aw2 = attw2_ref[...]

    def att_score(zz):
        t = jnp.tanh(jnp.dot(zz, aw1, preferred_element_type=jnp.float32) + ab1)
        return jnp.sum(t * aw2, axis=1, keepdims=True)

    z1 = e1
    z2 = com * np.float32(1.0 / 3.0)
    z3 = e2
    z4 = e3
    w1 = att_score(z1)
    w2 = att_score(z2)
    w3 = att_score(z3)
    w4 = att_score(z4)
    m = jnp.maximum(jnp.maximum(w1, w2), jnp.maximum(w3, w4))
    x1 = jnp.exp(w1 - m)
    x2 = jnp.exp(w2 - m)
    x3 = jnp.exp(w3 - m)
    x4 = jnp.exp(w4 - m)
    s = x1 + x2 + x3 + x4
    emb = (x1 * z1 + x2 * z2 + x3 * z3 + x4 * z4) / s
    emb = jnp.dot(emb, mlpw_ref[...], preferred_element_type=jnp.float32) + mlpb_ref[...]
    emb_ref[...] = emb

    h = jnp.dot(emb, decw_ref[...], preferred_element_type=jnp.float32) + decb_ref[...]
    h = jnp.maximum(h * _BN_SCALE * decg_ref[...] + decbeta_ref[...], 0.0)
    pi_ref[...] = jax.nn.sigmoid(
        jnp.dot(h, piw_ref[...], preferred_element_type=jnp.float32) + pib_ref[...])
    dv = jnp.dot(h, dispw_ref[...], preferred_element_type=jnp.float32) + dispb_ref[...]
    disp_ref[...] = jnp.clip(jax.nn.softplus(dv), 0.0001, 10000.0)
    mv = jnp.dot(h, meanw_ref[...], preferred_element_type=jnp.float32) + meanb_ref[...]
    mean_ref[...] = jnp.clip(jnp.exp(mv), 1e-05, 1000000.0)

    cpw = cpw_ref[...]
    cpb = cpb_ref[...]
    cpg = cpg_ref[...]
    cpbeta = cpbeta_ref[...]
    ccw = ccw_ref[...]
    ccb = ccb_ref[...]

    def cluster(v, y_out, p_out):
        zz = jnp.dot(v, cpw, preferred_element_type=jnp.float32) + cpb
        zz = jnp.maximum(zz * _BN_SCALE * cpg + cpbeta, 0.0)
        logits = jnp.dot(zz, ccw, preferred_element_type=jnp.float32) + ccb
        logits = logits - jnp.max(logits, axis=1, keepdims=True)
        ex = jnp.exp(logits)
        y_out[...] = ex / jnp.sum(ex, axis=1, keepdims=True)
        p_out[...] = zz

    cluster(c1_ref[...], y1_ref, p1_ref)
    cluster(c2_ref[...], y2_ref, p2_ref)
    cluster(c3_ref[...], y3_ref, p3_ref)
    cluster(com, y4_ref, p4_ref)
    cluster(emb, y_ref, pz_ref)


def kernel(x, x_aug, sadj, fadj, stg, params):
    p = params
    n, nfeat = x.shape

    def r2(v):
        return v.reshape(1, -1)

    ws = jnp.concatenate([p['sgcn_W1'], p['cgcn_W1']], axis=1)
    wf = jnp.concatenate([p['fgcn_W1'], p['cgcn_W1']], axis=1)
    wa = jnp.concatenate([p['agcn_W1'], p['cgcn_W1']], axis=1)
    h1 = p['sgcn_W1'].shape[1]

    def bd(wb, wc):
        z = jnp.zeros((2 * h1, 64), jnp.float32)
        z = z.at[:h1, :32].set(wb)
        z = z.at[h1:, 32:].set(wc)
        return z

    w2_s = bd(p['sgcn_W2'], p['cgcn_W2'])
    w2_f = bd(p['fgcn_W2'], p['cgcn_W2'])
    w2_a = bd(p['agcn_W2'], p['cgcn_W2'])
    b1_s = r2(jnp.concatenate([p['sgcn_b1'], p['cgcn_b1']]))
    b1_f = r2(jnp.concatenate([p['fgcn_b1'], p['cgcn_b1']]))
    b1_a = r2(jnp.concatenate([p['agcn_b1'], p['cgcn_b1']]))
    b2_s = r2(jnp.concatenate([p['sgcn_b2'], p['cgcn_b2']]))
    b2_f = r2(jnp.concatenate([p['fgcn_b2'], p['cgcn_b2']]))
    b2_a = r2(jnp.concatenate([p['agcn_b2'], p['cgcn_b2']]))

    hcat = 2 * h1
    ps, pf, q = pl.pallas_call(
        _proj_body,
        grid=(n // _BM_TAIL,),
        in_specs=[
            pl.BlockSpec((_BM_TAIL, nfeat), lambda i: (i, 0)),
            pl.BlockSpec((_BM_TAIL, nfeat), lambda i: (i, 0)),
            pl.BlockSpec((nfeat, hcat), lambda i: (0, 0)),
            pl.BlockSpec((nfeat, hcat), lambda i: (0, 0)),
            pl.BlockSpec((nfeat, hcat), lambda i: (0, 0)),
        ],
        out_specs=[pl.BlockSpec((_BM_TAIL, hcat), lambda i: (i, 0))] * 3,
        out_shape=[jax.ShapeDtypeStruct((n, hcat), jnp.float32)] * 3,
    )(x, x_aug, ws, wf, wa)

    emb1, com1 = _gcn_pair(sadj, ps, b1_s, w2_s, b2_s)
    emb2, com2 = _gcn_pair(fadj, pf, b1_f, w2_f, b2_f)
    emb3, com3 = _gcn_pair(stg, q, b1_a, w2_a, b2_a)

    nclass = p['cc_W'].shape[1]
    row_spec = lambda w: pl.BlockSpec((_BM_TAIL, w), lambda i: (i, 0))
    const_spec = lambda a: pl.BlockSpec(a.shape, lambda i: (0,) * a.ndim)

    weights = [
        p['att_W1'], r2(p['att_b1']), r2(p['att_W2'][:, 0]),
        p['mlp_W'], r2(p['mlp_b']),
        p['dec_W'], r2(p['dec_b']), r2(p['dec_g']), r2(p['dec_beta']),
        p['pi_W'], r2(p['pi_b']), p['disp_W'], r2(p['disp_b']),
        p['mean_W'], r2(p['mean_b']),
        p['cp_W'], r2(p['cp_b']), r2(p['cp_g']), r2(p['cp_beta']),
        p['cc_W'], r2(p['cc_b']),
    ]
    outs = pl.pallas_call(
        _tail_body,
        grid=(n // _BM_TAIL,),
        in_specs=[row_spec(32)] * 6 + [const_spec(w) for w in weights],
        out_specs=(
            [row_spec(32)] + [row_spec(nfeat)] * 3
            + [row_spec(nclass), row_spec(32)] * 5
        ),
        out_shape=(
            [jax.ShapeDtypeStruct((n, 32), jnp.float32)]
            + [jax.ShapeDtypeStruct((n, nfeat), jnp.float32)] * 3
            + [jax.ShapeDtypeStruct((n, nclass), jnp.float32),
               jax.ShapeDtypeStruct((n, 32), jnp.float32)] * 5
        ),
    )(emb1, emb2, emb3, com1, com2, com3, *weights)
    (emb, pi, disp, mean, y1, p1, y2, p2, y3, p3, y4, p4, y, pz) = outs
    return (com1, com2, com3, emb, pi, disp, mean,
            y1, p1, y2, p2, y3, p3, y4, p4, y, pz)


# speedup vs baseline: 1.0502x; 1.0159x over previous
import jax
import jax.numpy as jnp
import numpy as np
from jax.experimental import pallas as pl
from jax.experimental.pallas import tpu as pltpu

_BN_SCALE = np.float32(1.0 / np.sqrt(1.0 + 1e-5))

_BM = 400
_BM_TAIL = 2000


def _proj_body(x_ref, xa_ref, ws_ref, wf_ref, wa_ref, ps_ref, pf_ref, q_ref):
    x = x_ref[...]
    ps_ref[...] = jnp.dot(x, ws_ref[...], preferred_element_type=jnp.float32)
    pf_ref[...] = jnp.dot(x, wf_ref[...], preferred_element_type=jnp.float32)
    q_ref[...] = jnp.dot(xa_ref[...], wa_ref[...], preferred_element_type=jnp.float32)


def _gcn_body(adj_ref, p_ref, b1_ref, w2_ref, b2_ref, emb_ref, com_ref, u_scr):
    ph = pl.program_id(0)
    i = pl.program_id(1)

    @pl.when(ph == 0)
    def _layer1():
        h = jnp.dot(adj_ref[...], p_ref[...], preferred_element_type=jnp.float32)
        h = jnp.maximum(h + b1_ref[...], 0.0)
        u_scr[pl.ds(i * _BM, _BM), :] = jnp.dot(
            h, w2_ref[...], preferred_element_type=jnp.float32)

    @pl.when(ph == 1)
    def _layer2():
        o = jnp.dot(adj_ref[...], u_scr[...], preferred_element_type=jnp.float32)
        o = o + b2_ref[...]
        emb_ref[...] = o[:, :32]
        com_ref[...] = o[:, 32:]


def _gcn_pair(adj, p, b1cat, w2bd, b2cat):
    n = adj.shape[0]
    hcat = p.shape[1]
    nsteps = n // _BM

    def rowmap(ph, i):
        return (jnp.where(ph == 0, i, nsteps - 1 - i), 0)

    emb, com = pl.pallas_call(
        _gcn_body,
        grid=(2, nsteps),
        in_specs=[
            pl.BlockSpec((_BM, n), rowmap),
            pl.BlockSpec((n, hcat), lambda ph, i: (0, 0)),
            pl.BlockSpec((1, hcat), lambda ph, i: (0, 0)),
            pl.BlockSpec((hcat, 64), lambda ph, i: (0, 0)),
            pl.BlockSpec((1, 64), lambda ph, i: (0, 0)),
        ],
        out_specs=[
            pl.BlockSpec((_BM, 32), rowmap),
            pl.BlockSpec((_BM, 32), rowmap),
        ],
        out_shape=[
            jax.ShapeDtypeStruct((n, 32), jnp.float32),
            jax.ShapeDtypeStruct((n, 32), jnp.float32),
        ],
        scratch_shapes=[pltpu.VMEM((n, 64), jnp.float32)],
    )(adj, p, b1cat, w2bd, b2cat)
    return emb, com


def _head(e1, e2, e3, c1, c2, c3,
          attw1_ref, attb1_ref, attw2_ref,
          mlpw_ref, mlpb_ref,
          decw_ref, decb_ref, decg_ref, decbeta_ref,
          piw_ref, pib_ref, dispw_ref, dispb_ref, meanw_ref, meanb_ref,
          cpw_ref, cpb_ref, cpg_ref, cpbeta_ref, ccw_ref, ccb_ref,
          emb_ref, pi_ref, disp_ref, mean_ref,
          y1_ref, p1_ref, y2_ref, p2_ref, y3_ref, p3_ref,
          y4_ref, p4_ref, y_ref, pz_ref):
    com = e1 + e2 + e3

    aw1 = attw1_ref[...]
    ab1 = attb1_ref[...]
    aw2 = attw2_ref[...]

    def att_score(zz):
        t = jnp.tanh(jnp.dot(zz, aw1, preferred_element_type=jnp.float32) + ab1)
        return jnp.sum(t * aw2, axis=1, keepdims=True)

    z1 = e1
    z2 = com * np.float32(1.0 / 3.0)
    z3 = e2
    z4 = e3
    w1 = att_score(z1)
    w2 = att_score(z2)
    w3 = att_score(z3)
    w4 = att_score(z4)
    m = jnp.maximum(jnp.maximum(w1, w2), jnp.maximum(w3, w4))
    x1 = jnp.exp(w1 - m)
    x2 = jnp.exp(w2 - m)
    x3 = jnp.exp(w3 - m)
    x4 = jnp.exp(w4 - m)
    s = x1 + x2 + x3 + x4
    emb = (x1 * z1 + x2 * z2 + x3 * z3 + x4 * z4) / s
    emb = jnp.dot(emb, mlpw_ref[...], preferred_element_type=jnp.float32) + mlpb_ref[...]
    emb_ref[...] = emb

    h = jnp.dot(emb, decw_ref[...], preferred_element_type=jnp.float32) + decb_ref[...]
    h = jnp.maximum(h * _BN_SCALE * decg_ref[...] + decbeta_ref[...], 0.0)
    pi_ref[...] = jax.nn.sigmoid(
        jnp.dot(h, piw_ref[...], preferred_element_type=jnp.float32) + pib_ref[...])
    dv = jnp.dot(h, dispw_ref[...], preferred_element_type=jnp.float32) + dispb_ref[...]
    disp_ref[...] = jnp.clip(jax.nn.softplus(dv), 0.0001, 10000.0)
    mv = jnp.dot(h, meanw_ref[...], preferred_element_type=jnp.float32) + meanb_ref[...]
    mean_ref[...] = jnp.clip(jnp.exp(mv), 1e-05, 1000000.0)

    cpw = cpw_ref[...]
    cpb = cpb_ref[...]
    cpg = cpg_ref[...]
    cpbeta = cpbeta_ref[...]
    ccw = ccw_ref[...]
    ccb = ccb_ref[...]

    def cluster(v, y_out, p_out):
        zz = jnp.dot(v, cpw, preferred_element_type=jnp.float32) + cpb
        zz = jnp.maximum(zz * _BN_SCALE * cpg + cpbeta, 0.0)
        logits = jnp.dot(zz, ccw, preferred_element_type=jnp.float32) + ccb
        logits = logits - jnp.max(logits, axis=1, keepdims=True)
        ex = jnp.exp(logits)
        y_out[...] = ex / jnp.sum(ex, axis=1, keepdims=True)
        p_out[...] = zz

    cluster(c1, y1_ref, p1_ref)
    cluster(c2, y2_ref, p2_ref)
    cluster(c3, y3_ref, p3_ref)
    cluster(com, y4_ref, p4_ref)
    cluster(emb, y_ref, pz_ref)


def _gcn_head_body(adj_ref, p_ref, b1_ref, w2_ref, b2_ref,
                   e1_ref, c1_ref, e2_ref, c2_ref,
                   attw1_ref, attb1_ref, attw2_ref,
                   mlpw_ref, mlpb_ref,
                   decw_ref, decb_ref, decg_ref, decbeta_ref,
                   piw_ref, pib_ref, dispw_ref, dispb_ref,
                   meanw_ref, meanb_ref,
                   cpw_ref, cpb_ref, cpg_ref, cpbeta_ref, ccw_ref, ccb_ref,
                   c3_ref, emb_ref, pi_ref, disp_ref, mean_ref,
                   y1_ref, p1_ref, y2_ref, p2_ref, y3_ref, p3_ref,
                   y4_ref, p4_ref, y_ref, pz_ref, u_scr):
    ph = pl.program_id(0)
    i = pl.program_id(1)

    @pl.when(ph == 0)
    def _layer1():
        h = jnp.dot(adj_ref[...], p_ref[...], preferred_element_type=jnp.float32)
        h = jnp.maximum(h + b1_ref[...], 0.0)
        u_scr[pl.ds(i * _BM, _BM), :] = jnp.dot(
            h, w2_ref[...], preferred_element_type=jnp.float32)

    @pl.when(ph == 1)
    def _layer2_and_head():
        o = jnp.dot(adj_ref[...], u_scr[...], preferred_element_type=jnp.float32)
        o = o + b2_ref[...]
        e3 = o[:, :32]
        c3 = o[:, 32:]
        c3_ref[...] = c3
        _head(e1_ref[...], e2_ref[...], e3, c1_ref[...], c2_ref[...], c3,
              attw1_ref, attb1_ref, attw2_ref, mlpw_ref, mlpb_ref,
              decw_ref, decb_ref, decg_ref, decbeta_ref,
              piw_ref, pib_ref, dispw_ref, dispb_ref, meanw_ref, meanb_ref,
              cpw_ref, cpb_ref, cpg_ref, cpbeta_ref, ccw_ref, ccb_ref,
              emb_ref, pi_ref, disp_ref, mean_ref,
              y1_ref, p1_ref, y2_ref, p2_ref, y3_ref, p3_ref,
              y4_ref, p4_ref, y_ref, pz_ref)


def kernel(x, x_aug, sadj, fadj, stg, params):
    p = params
    n, nfeat = x.shape

    def r2(v):
        return v.reshape(1, -1)

    ws = jnp.concatenate([p['sgcn_W1'], p['cgcn_W1']], axis=1)
    wf = jnp.concatenate([p['fgcn_W1'], p['cgcn_W1']], axis=1)
    wa = jnp.concatenate([p['agcn_W1'], p['cgcn_W1']], axis=1)
    h1 = p['sgcn_W1'].shape[1]

    def bd(wb, wc):
        z = jnp.zeros((2 * h1, 64), jnp.float32)
        z = z.at[:h1, :32].set(wb)
        z = z.at[h1:, 32:].set(wc)
        return z

    w2_s = bd(p['sgcn_W2'], p['cgcn_W2'])
    w2_f = bd(p['fgcn_W2'], p['cgcn_W2'])
    w2_a = bd(p['agcn_W2'], p['cgcn_W2'])
    b1_s = r2(jnp.concatenate([p['sgcn_b1'], p['cgcn_b1']]))
    b1_f = r2(jnp.concatenate([p['fgcn_b1'], p['cgcn_b1']]))
    b1_a = r2(jnp.concatenate([p['agcn_b1'], p['cgcn_b1']]))
    b2_s = r2(jnp.concatenate([p['sgcn_b2'], p['cgcn_b2']]))
    b2_f = r2(jnp.concatenate([p['fgcn_b2'], p['cgcn_b2']]))
    b2_a = r2(jnp.concatenate([p['agcn_b2'], p['cgcn_b2']]))

    hcat = 2 * h1
    ps, pf, q = pl.pallas_call(
        _proj_body,
        grid=(n // _BM_TAIL,),
        in_specs=[
            pl.BlockSpec((_BM_TAIL, nfeat), lambda i: (i, 0)),
            pl.BlockSpec((_BM_TAIL, nfeat), lambda i: (i, 0)),
            pl.BlockSpec((nfeat, hcat), lambda i: (0, 0)),
            pl.BlockSpec((nfeat, hcat), lambda i: (0, 0)),
            pl.BlockSpec((nfeat, hcat), lambda i: (0, 0)),
        ],
        out_specs=[pl.BlockSpec((_BM_TAIL, hcat), lambda i: (i, 0))] * 3,
        out_shape=[jax.ShapeDtypeStruct((n, hcat), jnp.float32)] * 3,
    )(x, x_aug, ws, wf, wa)

    emb1, com1 = _gcn_pair(sadj, ps, b1_s, w2_s, b2_s)
    emb2, com2 = _gcn_pair(fadj, pf, b1_f, w2_f, b2_f)

    nclass = p['cc_W'].shape[1]
    nsteps = n // _BM
    last = nsteps - 1

    def phase1map(ph, i):
        return (jnp.where(ph == 0, last, last - i), 0)

    def adjmap(ph, i):
        return (jnp.where(ph == 0, i, last - i), 0)

    weights = [
        p['att_W1'], r2(p['att_b1']), r2(p['att_W2'][:, 0]),
        p['mlp_W'], r2(p['mlp_b']),
        p['dec_W'], r2(p['dec_b']), r2(p['dec_g']), r2(p['dec_beta']),
        p['pi_W'], r2(p['pi_b']), p['disp_W'], r2(p['disp_b']),
        p['mean_W'], r2(p['mean_b']),
        p['cp_W'], r2(p['cp_b']), r2(p['cp_g']), r2(p['cp_beta']),
        p['cc_W'], r2(p['cc_b']),
    ]
    const_spec = lambda a: pl.BlockSpec(a.shape, lambda ph, i: (0,) * a.ndim)
    row_spec = lambda w: pl.BlockSpec((_BM, w), phase1map)

    outs = pl.pallas_call(
        _gcn_head_body,
        grid=(2, nsteps),
        in_specs=(
            [
                pl.BlockSpec((_BM, n), adjmap),
                pl.BlockSpec((n, hcat), lambda ph, i: (0, 0)),
                pl.BlockSpec((1, hcat), lambda ph, i: (0, 0)),
                pl.BlockSpec((hcat, 64), lambda ph, i: (0, 0)),
                pl.BlockSpec((1, 64), lambda ph, i: (0, 0)),
            ]
            + [row_spec(32)] * 4
            + [const_spec(w) for w in weights]
        ),
        out_specs=(
            [row_spec(32), row_spec(32)] + [row_spec(nfeat)] * 3
            + [row_spec(nclass), row_spec(32)] * 5
        ),
        out_shape=(
            [jax.ShapeDtypeStruct((n, 32), jnp.float32)] * 2
            + [jax.ShapeDtypeStruct((n, nfeat), jnp.float32)] * 3
            + [jax.ShapeDtypeStruct((n, nclass), jnp.float32),
               jax.ShapeDtypeStruct((n, 32), jnp.float32)] * 5
        ),
        scratch_shapes=[pltpu.VMEM((n, 64), jnp.float32)],
    )(stg, q, b1_a, w2_a, b2_a, emb1, com1, emb2, com2, *weights)
    (com3, emb, pi, disp, mean,
     y1, p1, y2, p2, y3, p3, y4, p4, y, pz) = outs
    return (com1, com2, com3, emb, pi, disp, mean,
            y1, p1, y2, p2, y3, p3, y4, p4, y, pz)
